# baseline (speedup 1.0000x reference)
import jax
import jax.numpy as jnp
from jax.experimental import pallas as pl


def _body(a_ref, b_ref, o_ref):
    a = a_ref[...]
    o_ref[...] = jnp.where(a == 1.0, 0.0, a) + b_ref[...]


def kernel(x_1, x_2):
    n = x_1.size
    a = x_1.reshape(n)
    b = x_2.reshape(n)
    bn = 1 << 20
    out = pl.pallas_call(
        _body,
        grid=(n // bn,),
        in_specs=[
            pl.BlockSpec((bn,), lambda i: (i,)),
            pl.BlockSpec((bn,), lambda i: (i,)),
        ],
        out_specs=pl.BlockSpec((bn,), lambda i: (i,)),
        out_shape=jax.ShapeDtypeStruct((n,), jnp.float32),
    )(a, b)
    return out.reshape(x_1.shape)
